# baseline (speedup 1.0000x reference)
import functools

import jax
import jax.numpy as jnp
from jax import lax
from jax.experimental import pallas as pl
from jax.experimental.pallas import tpu as pltpu
from jax.experimental.pallas import tpu_sc as plsc

_NUM_LAYERS = 12
_PER_LAYER_DIM = 64
_ROW = _NUM_LAYERS * _PER_LAYER_DIM
_SEQ = 2048
_NUM_WORKERS = 32
_LANES = 16
_CHUNK = 16
_SCALE = float(_PER_LAYER_DIM) ** 0.5

_mesh = plsc.VectorSubcoreMesh(core_axis_name="c", subcore_axis_name="s")


def _make_gather_scale(seq):
    bpw = seq // _NUM_WORKERS
    nch = bpw // _CHUNK

    @functools.partial(
        pl.kernel,
        mesh=_mesh,
        out_type=jax.ShapeDtypeStruct((seq, _ROW), jnp.float32),
        scratch_types=[
            pltpu.VMEM((bpw,), jnp.int32),
            pltpu.VMEM((bpw, _ROW), jnp.float32),
        ]
        + [pltpu.SemaphoreType.DMA] * nch
        + [pltpu.SemaphoreType.DMA],
    )
    def _gather_scale(ids_hbm, table_hbm, out_hbm, idx_v, rows_v, *sems):
        gsems, osem = sems[:nch], sems[nch]
        wid = lax.axis_index("s") * 2 + lax.axis_index("c")
        base = wid * bpw
        pltpu.sync_copy(ids_hbm.at[pl.ds(base, bpw)], idx_v)

        gathers = []
        for g in range(nch):
            cp = pltpu.make_async_copy(
                table_hbm.at[idx_v.at[pl.ds(g * _CHUNK, _CHUNK)]],
                rows_v.at[pl.ds(g * _CHUNK, _CHUNK)],
                gsems[g],
            )
            cp.start()
            gathers.append(cp)

        copyouts = []
        for g in range(nch):
            gathers[g].wait()

            def scale_row(i, _):
                for j in range(_ROW // _LANES):
                    sl = pl.ds(j * _LANES, _LANES)
                    rows_v[i, sl] = rows_v[i, sl] * _SCALE
                return ()

            lax.fori_loop(g * _CHUNK, (g + 1) * _CHUNK, scale_row, (), unroll=False)
            out = pltpu.make_async_copy(
                rows_v.at[pl.ds(g * _CHUNK, _CHUNK)],
                out_hbm.at[pl.ds(base + g * _CHUNK, _CHUNK)],
                osem,
            )
            out.start()
            copyouts.append(out)

        for out in copyouts:
            out.wait()

    return _gather_scale


_HALF = _SEQ // 2
_gather_half = _make_gather_scale(_HALF)


def kernel(token_ids, per_layer_table):
    ids = token_ids.reshape(-1)
    b, s = token_ids.shape
    lo = _gather_half(ids[:_HALF], per_layer_table)
    hi = _gather_half(ids[_HALF:], per_layer_table)
    lo4 = lo.reshape(b, _HALF, _NUM_LAYERS, _PER_LAYER_DIM)
    hi4 = hi.reshape(b, _HALF, _NUM_LAYERS, _PER_LAYER_DIM)
    return jnp.concatenate([lo4, hi4], axis=1)


# speedup vs baseline: 1.0512x; 1.0512x over previous
import functools

import jax
import jax.numpy as jnp
from jax import lax
from jax.experimental import pallas as pl
from jax.experimental.pallas import tpu as pltpu
from jax.experimental.pallas import tpu_sc as plsc

_NUM_LAYERS = 12
_PER_LAYER_DIM = 64
_ROW = _NUM_LAYERS * _PER_LAYER_DIM
_SEQ = 2048
_NUM_WORKERS = 32
_BPW = _SEQ // _NUM_WORKERS
_CHUNK = 16
_NCH = _BPW // _CHUNK
_SCALE = float(_PER_LAYER_DIM) ** 0.5

_mesh = plsc.VectorSubcoreMesh(core_axis_name="c", subcore_axis_name="s")


@functools.partial(
    pl.kernel,
    mesh=_mesh,
    out_type=jax.ShapeDtypeStruct((_SEQ, _ROW), jnp.float32),
    scratch_types=[
        pltpu.VMEM((_BPW,), jnp.int32),
        pltpu.VMEM((_BPW, _ROW), jnp.float32),
    ]
    + [pltpu.SemaphoreType.DMA] * _NCH
    + [pltpu.SemaphoreType.DMA],
)
def _gather_rows(ids_hbm, table_hbm, out_hbm, idx_v, rows_v, *sems):
    gsems, osem = sems[:_NCH], sems[_NCH]
    wid = lax.axis_index("s") * 2 + lax.axis_index("c")
    base = wid * _BPW
    pltpu.sync_copy(ids_hbm.at[pl.ds(base, _BPW)], idx_v)

    gathers = []
    for g in range(_NCH):
        cp = pltpu.make_async_copy(
            table_hbm.at[idx_v.at[pl.ds(g * _CHUNK, _CHUNK)]],
            rows_v.at[pl.ds(g * _CHUNK, _CHUNK)],
            gsems[g],
        )
        cp.start()
        gathers.append(cp)

    copyouts = []
    for g in range(_NCH):
        gathers[g].wait()
        out = pltpu.make_async_copy(
            rows_v.at[pl.ds(g * _CHUNK, _CHUNK)],
            out_hbm.at[pl.ds(base + g * _CHUNK, _CHUNK)],
            osem,
        )
        out.start()
        copyouts.append(out)

    for out in copyouts:
        out.wait()


def kernel(token_ids, per_layer_table):
    ids = token_ids.reshape(-1)
    rows = _gather_rows(ids, per_layer_table)
    b, s = token_ids.shape
    emb = rows.reshape(b, s, _NUM_LAYERS, _PER_LAYER_DIM)
    return emb * jnp.float32(_SCALE)


# speedup vs baseline: 1.2189x; 1.1595x over previous
import functools

import jax
import jax.numpy as jnp
from jax import lax
from jax.experimental import pallas as pl
from jax.experimental.pallas import tpu as pltpu
from jax.experimental.pallas import tpu_sc as plsc

_NUM_LAYERS = 12
_PER_LAYER_DIM = 64
_ROW = _NUM_LAYERS * _PER_LAYER_DIM
_SEQ = 2048
_NUM_WORKERS = 32
_BPW = _SEQ // _NUM_WORKERS
_LANES = 16
_CHUNK = 16
_NCH = _BPW // _CHUNK
_SCALE = float(_PER_LAYER_DIM) ** 0.5

_mesh = plsc.VectorSubcoreMesh(core_axis_name="c", subcore_axis_name="s")


@functools.partial(
    pl.kernel,
    mesh=_mesh,
    out_type=jax.ShapeDtypeStruct((_SEQ, _ROW), jnp.float32),
    scratch_types=[
        pltpu.VMEM((_BPW,), jnp.int32),
        pltpu.VMEM((_BPW, _ROW), jnp.float32),
    ]
    + [pltpu.SemaphoreType.DMA] * _NCH
    + [pltpu.SemaphoreType.DMA],
)
def _gather_scale(ids_hbm, table_hbm, out_hbm, idx_v, rows_v, *sems):
    gsems, osem = sems[:_NCH], sems[_NCH]
    wid = lax.axis_index("s") * 2 + lax.axis_index("c")
    base = wid * _BPW
    pltpu.sync_copy(ids_hbm.at[pl.ds(base, _BPW)], idx_v)

    gathers = []
    for g in range(_NCH):
        cp = pltpu.make_async_copy(
            table_hbm.at[idx_v.at[pl.ds(g * _CHUNK, _CHUNK)]],
            rows_v.at[pl.ds(g * _CHUNK, _CHUNK)],
            gsems[g],
        )
        cp.start()
        gathers.append(cp)

    copyouts = []
    for g in range(_NCH):
        gathers[g].wait()

        def scale_row(i, _):
            for j in range(_ROW // _LANES):
                sl = pl.ds(j * _LANES, _LANES)
                rows_v[i, sl] = rows_v[i, sl] * _SCALE
            return ()

        lax.fori_loop(g * _CHUNK, (g + 1) * _CHUNK, scale_row, (), unroll=False)
        out = pltpu.make_async_copy(
            rows_v.at[pl.ds(g * _CHUNK, _CHUNK)],
            out_hbm.at[pl.ds(base + g * _CHUNK, _CHUNK)],
            osem,
        )
        out.start()
        copyouts.append(out)

    for out in copyouts:
        out.wait()


def kernel(token_ids, per_layer_table):
    ids = token_ids.reshape(-1)
    out = _gather_scale(ids, per_layer_table)
    b, s = token_ids.shape
    return out.reshape(b, s, _NUM_LAYERS, _PER_LAYER_DIM)
